# Initial kernel scaffold; baseline (speedup 1.0000x reference)
import jax
import jax.numpy as jnp
from jax.experimental import pallas as pl


def kernel(x, bias):
    raise NotImplementedError("write your pallas kernel here")


# speedup vs baseline: 1306.2370x; 1306.2370x over previous
import functools

import jax
import jax.numpy as jnp
from jax import lax
from jax.experimental import pallas as pl
from jax.experimental.pallas import tpu as pltpu
from jax.experimental.pallas import tpu_sc as plsc

MAXREL = 128
TBL = 2 * MAXREL + 1
SEQ = 4096
NWORKERS = 32
ROWS_PER_W = SEQ // NWORKERS
NCOPY = 8
VLEN = 2 * SEQ + 16
CONST_OFF = SEQ - 1 - MAXREL
TBL_PAD = 272


def _tec_body(bias_hbm, out_hbm, bias_v, vcop, sem):
    cid = lax.axis_index("c")
    sid = lax.axis_index("s")
    wid = sid * 2 + cid

    pltpu.sync_copy(bias_hbm, bias_v)

    lanes = lax.iota(jnp.int32, 16)
    n_chunks = VLEN // 16

    def build(t, carry):
        cc = t // n_chunks
        k0 = (t % n_chunks) * 16
        idx = jnp.clip(k0 + lanes - cc - CONST_OFF, 0, TBL - 1)
        vcop[pl.ds(t * 16, 16)] = plsc.load_gather(bias_v, [idx])
        return carry

    lax.fori_loop(0, NCOPY * n_chunks, build, 0)

    base = wid * ROWS_PER_W

    def row(r, carry):
        i = base + r
        s = SEQ - 1 - i
        cc = (i + 1) % 8
        start = pl.multiple_of(cc * VLEN + s + cc, 8)
        pltpu.async_copy(vcop.at[pl.ds(start, SEQ)], out_hbm.at[i], sem).wait()
        return carry

    lax.fori_loop(0, ROWS_PER_W, row, 0)


def _build(bias_pad):
    mesh = plsc.VectorSubcoreMesh(core_axis_name="c", subcore_axis_name="s")
    kern = functools.partial(
        pl.kernel,
        mesh=mesh,
        out_type=jax.ShapeDtypeStruct((SEQ, SEQ), jnp.float32),
        scratch_types=[
            pltpu.VMEM((TBL_PAD,), jnp.float32),
            pltpu.VMEM((NCOPY * VLEN,), jnp.float32),
            pltpu.SemaphoreType.DMA,
        ],
        compiler_params=pltpu.CompilerParams(
            needs_layout_passes=False, use_tc_tiling_on_sc=False
        ),
    )(_tec_body)
    return kern(bias_pad)


def kernel(x, bias):
    del x
    bias_pad = jnp.zeros((TBL_PAD,), jnp.float32).at[:TBL].set(bias)
    return _build(bias_pad)


# speedup vs baseline: 1398.8261x; 1.0709x over previous
import functools

import jax
import jax.numpy as jnp
from jax import lax
from jax.experimental import pallas as pl
from jax.experimental.pallas import tpu as pltpu
from jax.experimental.pallas import tpu_sc as plsc

MAXREL = 128
TBL = 2 * MAXREL + 1
SEQ = 4096
NWORKERS = 32
ROWS_PER_W = SEQ // NWORKERS
NCOPY = 8
VLEN = 2 * SEQ + 32
CONST_OFF = SEQ - 1 - MAXREL
TBL_PAD = 272
NSUB = 16
CHUNKS_PER_SUB = NCOPY * VLEN // (16 * NSUB)


def _tec_body(bias_hbm, out_hbm, bias_v, stage, vfull, sem):
    cid = lax.axis_index("c")
    sid = lax.axis_index("s")
    wid = sid * 2 + cid

    pltpu.sync_copy(bias_hbm, bias_v)

    lanes = lax.iota(jnp.int32, 16)

    def build(t, carry):
        g = sid * CHUNKS_PER_SUB + t
        c = g // (VLEN // 16)
        k0 = (g % (VLEN // 16)) * 16
        idx = jnp.clip(k0 + lanes - c - CONST_OFF, 0, TBL - 1)
        stage[pl.ds(t * 16, 16)] = plsc.load_gather(bias_v, [idx])
        return carry

    lax.fori_loop(0, CHUNKS_PER_SUB, build, 0)
    pltpu.sync_copy(
        stage, vfull.at[pl.ds(sid * CHUNKS_PER_SUB * 16, CHUNKS_PER_SUB * 16)]
    )
    plsc.subcore_barrier()

    base = wid * ROWS_PER_W

    def _row_copy(r):
        i = base + r
        s = SEQ - 1 - i
        cc = (i + 1) % 8
        start = pl.multiple_of(cc * VLEN + s + cc, 8)
        return pltpu.make_async_copy(
            vfull.at[pl.ds(start, SEQ)], out_hbm.at[i], sem
        )

    def row_start(r, carry):
        _row_copy(r).start()
        return carry

    def row_wait(r, carry):
        _row_copy(r).wait()
        return carry

    lax.fori_loop(0, ROWS_PER_W, row_start, 0)
    lax.fori_loop(0, ROWS_PER_W, row_wait, 0)


def _build(bias_pad):
    mesh = plsc.VectorSubcoreMesh(core_axis_name="c", subcore_axis_name="s")
    kern = functools.partial(
        pl.kernel,
        mesh=mesh,
        out_type=jax.ShapeDtypeStruct((SEQ, SEQ), jnp.float32),
        scratch_types=[
            pltpu.VMEM((TBL_PAD,), jnp.float32),
            pltpu.VMEM((CHUNKS_PER_SUB * 16,), jnp.float32),
            pltpu.VMEM_SHARED((NCOPY * VLEN,), jnp.float32),
            pltpu.SemaphoreType.DMA,
        ],
        compiler_params=pltpu.CompilerParams(
            needs_layout_passes=False, use_tc_tiling_on_sc=False
        ),
    )(_tec_body)
    return kern(bias_pad)


def kernel(x, bias):
    del x
    bias_pad = jnp.zeros((TBL_PAD,), jnp.float32).at[:TBL].set(bias)
    return _build(bias_pad)


# speedup vs baseline: 2787.9657x; 1.9931x over previous
import functools

import jax
import jax.numpy as jnp
from jax import lax
from jax.experimental import pallas as pl
from jax.experimental.pallas import tpu as pltpu
from jax.experimental.pallas import tpu_sc as plsc

MAXREL = 128
TBL = 2 * MAXREL + 1
SEQ = 4096
NBLK = 16
VLEN = 8192
SHIFT0 = SEQ - 1 - MAXREL + 1
TBL_PAD = 272
CT_LO = 31
CT_HI = 33
CSW = 256


def _tec_body(bias_hbm, out_hbm, bias_v, cst0, cst1, stage, cseed0, cseed1,
              v2x, fsem, sem):
    cid = lax.axis_index("c")
    sid = lax.axis_index("s")

    pltpu.sync_copy(bias_hbm, bias_v)

    lanes = lax.iota(jnp.int32, 16)
    big = jnp.full((16,), 3.4e38, jnp.float32)
    v_lo = bias_v[pl.ds(0, 16)]
    v_hi = bias_v[pl.ds(TBL - 1, 16)]
    splat0 = jnp.full((16,), jnp.min(jnp.where(lanes == 0, v_lo, big)))
    splat1 = jnp.full((16,), jnp.min(jnp.where(lanes == 0, v_hi, big)))

    for r in range(8):
        for t in range(CSW // 16):
            cst0[r, pl.ds(t * 16, 16)] = splat0
            cst1[r, pl.ds(t * 16, 16)] = splat1
    stripe = pl.multiple_of(sid * CSW, CSW)
    pltpu.sync_copy(cst0, cseed0.at[:, pl.ds(stripe, CSW)])
    pltpu.sync_copy(cst1, cseed1.at[:, pl.ds(stripe, CSW)])
    plsc.subcore_barrier()

    @pl.when(sid < 8)
    def _build_slab():
        s8 = 16 * sid + 8 * cid
        f0 = pltpu.async_copy(
            cseed0.at[:, pl.ds(0, CT_LO * 128)],
            stage.at[:, pl.ds(0, CT_LO * 128)],
            fsem,
        )
        f1 = pltpu.async_copy(
            cseed1.at[:, pl.ds(0, (63 - CT_HI) * 128)],
            stage.at[:, pl.ds((CT_HI + 1) * 128, (63 - CT_HI) * 128)],
            fsem,
        )
        for ct in range(CT_LO, CT_HI + 1):
            for r in range(8):
                for t in range(8):
                    col = ct * 128 + t * 16
                    idx = jnp.clip(col + lanes - s8 - r - SHIFT0, 0, TBL - 1)
                    stage[r, pl.ds(col, 16)] = plsc.load_gather(bias_v, [idx])
        f0.wait()
        f1.wait()
        pltpu.sync_copy(stage, v2x.at[pl.ds(pl.multiple_of(8 * sid, 8), 8), :])

    plsc.subcore_barrier()

    trow = pl.multiple_of(8 * (sid % 8), 8)
    kbase = SEQ - 128 * (sid // 8)

    def _blk(b):
        i0 = pl.multiple_of(16 * sid + 8 * cid + 256 * b, 8)
        k0 = pl.multiple_of(kbase - 256 * b, 128)
        return pltpu.make_async_copy(
            v2x.at[pl.ds(trow, 8), pl.ds(k0, SEQ)],
            out_hbm.at[pl.ds(i0, 8), :],
            sem,
        )

    copies = [_blk(b) for b in range(NBLK)]
    for cp in copies:
        cp.start()
    for cp in copies:
        cp.wait()


def _build(bias_pad):
    mesh = plsc.VectorSubcoreMesh(core_axis_name="c", subcore_axis_name="s")
    kern = functools.partial(
        pl.kernel,
        mesh=mesh,
        out_type=jax.ShapeDtypeStruct((SEQ, SEQ), jnp.float32),
        scratch_types=[
            pltpu.VMEM((TBL_PAD,), jnp.float32),
            pltpu.VMEM((8, CSW), jnp.float32),
            pltpu.VMEM((8, CSW), jnp.float32),
            pltpu.VMEM((8, VLEN), jnp.float32),
            pltpu.VMEM_SHARED((8, 16 * CSW), jnp.float32),
            pltpu.VMEM_SHARED((8, 16 * CSW), jnp.float32),
            pltpu.VMEM_SHARED((64, VLEN), jnp.float32),
            pltpu.SemaphoreType.DMA,
            pltpu.SemaphoreType.DMA,
        ],
        compiler_params=pltpu.CompilerParams(needs_layout_passes=False),
    )(_tec_body)
    return kern(bias_pad)


def kernel(x, bias):
    del x
    bias_pad = jnp.zeros((TBL_PAD,), jnp.float32).at[:TBL].set(bias)
    return _build(bias_pad)
